# initial kernel scaffold (unmeasured)
import jax
import jax.numpy as jnp
from jax import lax
from jax.experimental import pallas as pl
from jax.experimental.pallas import tpu as pltpu

N_DEV = 32


def kernel(x, Win0, Wout0, Win1, Wout1, Win2, Wout2):
    b, _ = x.shape
    h = Win0.shape[1]
    rows = b // N_DEV

    def body(x_ref, win0_ref, wout0_ref, win1_ref, wout1_ref, win2_ref,
             wout2_ref, out_ref, part_ref, hme_ref, rs_ref, ag_ref,
             rs_send_sems, rs_recv_sems, ag_send_sems, ag_recv_sems):
        my = lax.axis_index("i")
        win_refs = [win0_ref, win1_ref, win2_ref]
        wout_refs = [wout0_ref, wout1_ref, wout2_ref]

        x_val = x_ref[...]

        for layer in range(3):
            part = jnp.dot(x_val, win_refs[layer][...],
                           preferred_element_type=jnp.float32)
            part_ref[...] = part

            for p in range(N_DEV):
                @pl.when(my != p)
                def _(p=p):
                    pltpu.make_async_remote_copy(
                        src_ref=part_ref.at[pl.ds(p * rows, rows), :],
                        dst_ref=rs_ref.at[my],
                        send_sem=rs_send_sems.at[p],
                        recv_sem=rs_recv_sems.at[my],
                        device_id=(p,),
                        device_id_type=pl.DeviceIdType.MESH,
                    ).start()
            rs_ref[my] = lax.dynamic_slice_in_dim(part, my * rows, rows, axis=0)

            for p in range(N_DEV):
                @pl.when(my != p)
                def _(p=p):
                    pltpu.make_async_remote_copy(
                        src_ref=part_ref.at[pl.ds(p * rows, rows), :],
                        dst_ref=rs_ref.at[p],
                        send_sem=rs_send_sems.at[p],
                        recv_sem=rs_recv_sems.at[p],
                        device_id=(p,),
                        device_id_type=pl.DeviceIdType.MESH,
                    ).wait_recv()

            h_me = jnp.maximum(jnp.sum(rs_ref[...], axis=0), 0.0)
            hme_ref[...] = h_me

            for p in range(N_DEV):
                @pl.when(my != p)
                def _(p=p):
                    pltpu.make_async_remote_copy(
                        src_ref=hme_ref,
                        dst_ref=ag_ref.at[my],
                        send_sem=ag_send_sems.at[p],
                        recv_sem=ag_recv_sems.at[my],
                        device_id=(p,),
                        device_id_type=pl.DeviceIdType.MESH,
                    ).start()
            ag_ref[my] = h_me

            for p in range(N_DEV):
                @pl.when(my != p)
                def _(p=p):
                    pltpu.make_async_remote_copy(
                        src_ref=hme_ref,
                        dst_ref=ag_ref.at[p],
                        send_sem=ag_send_sems.at[p],
                        recv_sem=ag_recv_sems.at[p],
                        device_id=(p,),
                        device_id_type=pl.DeviceIdType.MESH,
                    ).wait_recv()

            h_full = ag_ref[...].reshape(b, h)
            x_val = jnp.dot(h_full, wout_refs[layer][...],
                            preferred_element_type=jnp.float32)

            for p in range(N_DEV):
                @pl.when(my != p)
                def _(p=p):
                    pltpu.make_async_remote_copy(
                        src_ref=part_ref.at[pl.ds(p * rows, rows), :],
                        dst_ref=rs_ref.at[my],
                        send_sem=rs_send_sems.at[p],
                        recv_sem=rs_recv_sems.at[my],
                        device_id=(p,),
                        device_id_type=pl.DeviceIdType.MESH,
                    ).wait_send()
                    pltpu.make_async_remote_copy(
                        src_ref=hme_ref,
                        dst_ref=ag_ref.at[my],
                        send_sem=ag_send_sems.at[p],
                        recv_sem=ag_recv_sems.at[my],
                        device_id=(p,),
                        device_id_type=pl.DeviceIdType.MESH,
                    ).wait_send()

        out_ref[...] = x_val

    return pl.pallas_call(
        body,
        out_shape=jax.ShapeDtypeStruct(x.shape, jnp.float32),
        in_specs=[pl.BlockSpec(memory_space=pltpu.VMEM)] * 7,
        out_specs=pl.BlockSpec(memory_space=pltpu.VMEM),
        scratch_shapes=[
            pltpu.VMEM((b, h), jnp.float32),
            pltpu.VMEM((rows, h), jnp.float32),
            pltpu.VMEM((N_DEV, rows, h), jnp.float32),
            pltpu.VMEM((N_DEV, rows, h), jnp.float32),
            pltpu.SemaphoreType.DMA((N_DEV,)),
            pltpu.SemaphoreType.DMA((N_DEV,)),
            pltpu.SemaphoreType.DMA((N_DEV,)),
            pltpu.SemaphoreType.DMA((N_DEV,)),
        ],
    )(x, Win0, Wout0, Win1, Wout1, Win2, Wout2)


# baseline (device time: 72702 ns/iter reference)
import jax
import jax.numpy as jnp
from jax import lax
from jax.experimental import pallas as pl
from jax.experimental.pallas import tpu as pltpu

N_DEV = 32


def kernel(x, Win0, Wout0, Win1, Wout1, Win2, Wout2):
    b, _ = x.shape
    h = Win0.shape[1]
    rows = b // N_DEV

    def body(x_ref, win0_ref, wout0_ref, win1_ref, wout1_ref, win2_ref,
             wout2_ref, out_ref, part_ref, hme_ref, rs_ref, ag_ref,
             rs_send_sems, rs_recv_sems, ag_send_sems, ag_recv_sems):
        my = lax.axis_index("i")
        win_refs = [win0_ref, win1_ref, win2_ref]
        wout_refs = [wout0_ref, wout1_ref, wout2_ref]

        x_val = x_ref[...]

        for layer in range(3):
            part = jnp.dot(x_val, win_refs[layer][...],
                           preferred_element_type=jnp.float32)
            part_ref[...] = part

            for p in range(N_DEV):
                @pl.when(my != p)
                def _(p=p):
                    pltpu.make_async_remote_copy(
                        src_ref=part_ref.at[pl.ds(p * rows, rows), :],
                        dst_ref=rs_ref.at[my],
                        send_sem=rs_send_sems.at[p],
                        recv_sem=rs_recv_sems.at[my],
                        device_id=(p,),
                        device_id_type=pl.DeviceIdType.MESH,
                    ).start()
            rs_ref[my] = part_ref[pl.ds(my * rows, rows), :]

            for p in range(N_DEV):
                @pl.when(my != p)
                def _(p=p):
                    pltpu.make_async_remote_copy(
                        src_ref=part_ref.at[pl.ds(p * rows, rows), :],
                        dst_ref=rs_ref.at[p],
                        send_sem=rs_send_sems.at[p],
                        recv_sem=rs_recv_sems.at[p],
                        device_id=(p,),
                        device_id_type=pl.DeviceIdType.MESH,
                    ).wait_recv()

            h_me = jnp.maximum(jnp.sum(rs_ref[...], axis=0), 0.0)
            hme_ref[...] = h_me

            for p in range(N_DEV):
                @pl.when(my != p)
                def _(p=p):
                    pltpu.make_async_remote_copy(
                        src_ref=hme_ref,
                        dst_ref=ag_ref.at[my],
                        send_sem=ag_send_sems.at[p],
                        recv_sem=ag_recv_sems.at[my],
                        device_id=(p,),
                        device_id_type=pl.DeviceIdType.MESH,
                    ).start()
            ag_ref[my] = h_me

            for p in range(N_DEV):
                @pl.when(my != p)
                def _(p=p):
                    pltpu.make_async_remote_copy(
                        src_ref=hme_ref,
                        dst_ref=ag_ref.at[p],
                        send_sem=ag_send_sems.at[p],
                        recv_sem=ag_recv_sems.at[p],
                        device_id=(p,),
                        device_id_type=pl.DeviceIdType.MESH,
                    ).wait_recv()

            h_full = ag_ref[...].reshape(b, h)
            x_val = jnp.dot(h_full, wout_refs[layer][...],
                            preferred_element_type=jnp.float32)

            for p in range(N_DEV):
                @pl.when(my != p)
                def _(p=p):
                    pltpu.make_async_remote_copy(
                        src_ref=part_ref.at[pl.ds(p * rows, rows), :],
                        dst_ref=rs_ref.at[my],
                        send_sem=rs_send_sems.at[p],
                        recv_sem=rs_recv_sems.at[my],
                        device_id=(p,),
                        device_id_type=pl.DeviceIdType.MESH,
                    ).wait_send()
                    pltpu.make_async_remote_copy(
                        src_ref=hme_ref,
                        dst_ref=ag_ref.at[my],
                        send_sem=ag_send_sems.at[p],
                        recv_sem=ag_recv_sems.at[my],
                        device_id=(p,),
                        device_id_type=pl.DeviceIdType.MESH,
                    ).wait_send()

        out_ref[...] = x_val

    return pl.pallas_call(
        body,
        out_shape=jax.ShapeDtypeStruct(x.shape, jnp.float32),
        in_specs=[pl.BlockSpec(memory_space=pltpu.VMEM)] * 7,
        out_specs=pl.BlockSpec(memory_space=pltpu.VMEM),
        scratch_shapes=[
            pltpu.VMEM((b, h), jnp.float32),
            pltpu.VMEM((rows, h), jnp.float32),
            pltpu.VMEM((N_DEV, rows, h), jnp.float32),
            pltpu.VMEM((N_DEV, rows, h), jnp.float32),
            pltpu.SemaphoreType.DMA((N_DEV,)),
            pltpu.SemaphoreType.DMA((N_DEV,)),
            pltpu.SemaphoreType.DMA((N_DEV,)),
            pltpu.SemaphoreType.DMA((N_DEV,)),
        ],
    )(x, Win0, Wout0, Win1, Wout1, Win2, Wout2)
